# baseline (speedup 1.0000x reference)
import functools

import jax
import jax.numpy as jnp
from jax import lax
from jax.experimental import pallas as pl
from jax.experimental.pallas import tpu as pltpu
from jax.experimental.pallas import tpu_sc as plsc

L = 16
NW = 32
N_SAMPLES = 4096
N_CALLS = 4
N_SCALL = N_SAMPLES // N_CALLS
N_SITES = 100
N_ROWS = 2 * N_SITES
D = 128
SITES_PAD = 128
N_CHUNKS = 7
ROWS_BUF = 104
DCH = D // L
SPW = N_SCALL // NW
IDX_STRIDE = 104
NBUF = 4
G_ROWS = 48
G_GRP0 = G_ROWS // L


def _vfull(val):
    return jnp.full((L,), val, jnp.int32)


def _bcast_last(cs):
    return plsc.cummax(lax.rev(cs, (0,)))


def _sc_body(x_hbm, table_hbm, out_hbm, x_v, table_v, idx1d,
             rows0, rows1, rows2, rows3,
             wsem0, wsem1, wsem2, wsem3,
             gsem0, gsem1, gsem2, gsem3):
    rows = (rows0, rows1, rows2, rows3)
    wsems = (wsem0, wsem1, wsem2, wsem3)
    gsems = (gsem0, gsem1, gsem2, gsem3)

    wid = lax.axis_index("s") * 2 + lax.axis_index("c")
    base_s = wid * SPW

    pltpu.sync_copy(x_hbm.at[pl.ds(base_s, SPW)], x_v)
    pltpu.sync_copy(table_hbm, table_v)

    iota = lax.iota(jnp.int32, L)
    ones_v = _vfull(1)
    zeros_v = _vfull(0)
    negones_v = _vfull(-1)

    def build_sample(smp, smp_off):
        n_up = zeros_v
        for c in range(N_CHUNKS):
            v = x_v[smp, pl.ds(c * L, L)]
            upi = jnp.where(v == ones_v, ones_v, zeros_v)
            n_up = n_up + _bcast_last(plsc.cumsum(upi))
        carry = zeros_v
        for c in range(N_CHUNKS):
            v = x_v[smp, pl.ds(c * L, L)]
            up = v == ones_v
            dn = v == negones_v
            upi = jnp.where(up, ones_v, zeros_v)
            cs = plsc.cumsum(upi)
            up_ex = carry + cs - upi
            i_loc = iota + _vfull(c * L)
            p = jnp.where(up, up_ex, n_up + i_loc - up_ex)
            src = jnp.where(dn, i_loc + _vfull(N_SITES), i_loc)
            if c < 6:
                plsc.store_scatter(idx1d, [smp_off + p], src)
            else:
                plsc.store_scatter(idx1d, [smp_off + p], src,
                                   mask=i_loc < _vfull(IDX_STRIDE))
            carry = carry + _bcast_last(cs)

    def assemble_sample(smp, b):
        base = smp * IDX_STRIDE

        def grp(g, _):
            j0 = jnp.minimum(g * L, IDX_STRIDE - L)
            rv = idx1d[pl.ds(base + j0, L)]
            for l in range(L):
                r = rv[l]
                for c in range(DCH):
                    rows[b][j0 + l, pl.ds(c * L, L)] = \
                        table_v[r, pl.ds(c * L, L)]
            return 0

        lax.fori_loop(G_GRP0, N_CHUNKS, grp, 0)

    def gather(k, b):
        pltpu.async_copy(
            table_hbm.at[idx1d.at[pl.ds(k * IDX_STRIDE, G_ROWS)]],
            rows[b].at[pl.ds(0, G_ROWS)], gsems[b])

    def wait_gather(b):
        pltpu.make_async_copy(
            table_hbm.at[idx1d.at[pl.ds(0, G_ROWS)]],
            rows[b].at[pl.ds(0, G_ROWS)], gsems[b]).wait()

    def write(k, b):
        pltpu.async_copy(
            rows[b].at[pl.ds(0, N_SITES)], out_hbm.at[base_s + k], wsems[b])

    def wait_write(b):
        pltpu.make_async_copy(
            rows[b].at[pl.ds(0, N_SITES)], out_hbm.at[base_s], wsems[b]).wait()

    def move(g, off0):
        for b in range(NBUF):
            k = g * NBUF + b
            build_sample(k, off0 + _vfull(b * IDX_STRIDE))

            @pl.when(g >= 1)
            def _():
                wait_write(b)

            gather(k, b)
            assemble_sample(k, b)
            wait_gather(b)
            write(k, b)
        return off0 + _vfull(NBUF * IDX_STRIDE)

    lax.fori_loop(0, SPW // NBUF, move, zeros_v)

    for b in range(NBUF):
        wait_write(b)


_sc_kernel = functools.partial(
    pl.kernel,
    out_type=jax.ShapeDtypeStruct((N_SCALL, N_SITES, D), jnp.float32),
    mesh=plsc.VectorSubcoreMesh(core_axis_name="c", subcore_axis_name="s"),
    compiler_params=pltpu.CompilerParams(
        needs_layout_passes=False, use_tc_tiling_on_sc=True),
    scratch_types=[
        pltpu.VMEM((SPW, SITES_PAD), jnp.int32),
        pltpu.VMEM((N_ROWS, D), jnp.float32),
        pltpu.VMEM((SPW * IDX_STRIDE + L,), jnp.int32),
    ]
    + [pltpu.VMEM((ROWS_BUF, D), jnp.float32) for _ in range(NBUF)]
    + [pltpu.SemaphoreType.DMA for _ in range(2 * NBUF)],
)(_sc_body)


def kernel(x, orbitals_mf, orbitals_hf):
    n_samples, n_sites = x.shape
    assert (n_samples, n_sites) == (N_SAMPLES, N_SITES)
    table = jnp.concatenate([orbitals_mf, orbitals_hf], axis=1)
    xp = jnp.pad(x.astype(jnp.int32), ((0, 0), (0, SITES_PAD - n_sites)))
    outs = [_sc_kernel(xp[i * N_SCALL:(i + 1) * N_SCALL], table)
            for i in range(N_CALLS)]
    return jnp.concatenate(outs, axis=0)


# speedup vs baseline: 1.3948x; 1.3948x over previous
import functools

import jax
import jax.numpy as jnp
from jax import lax
from jax.experimental import pallas as pl
from jax.experimental.pallas import tpu as pltpu
from jax.experimental.pallas import tpu_sc as plsc

L = 16
NW = 32
N_SAMPLES = 4096
N_SITES = 100
N_ROWS = 2 * N_SITES
D = 128
SITES_PAD = 128
N_CHUNKS = 7
ROWS_BUF = 104
DCH = D // L
SPW = N_SAMPLES // NW
IDX_STRIDE = 104
NBUF = 4
G_ROWS = 64
G_GRP0 = G_ROWS // L


def _vfull(val):
    return jnp.full((L,), val, jnp.int32)


def _bcast_last(cs):
    return plsc.cummax(lax.rev(cs, (0,)))


def _sc_body(x_hbm, table_hbm, out_hbm, x_v, table_v, idx1d,
             rows0, rows1, rows2, rows3,
             wsem0, wsem1, wsem2, wsem3,
             gsem0, gsem1, gsem2, gsem3):
    rows = (rows0, rows1, rows2, rows3)
    wsems = (wsem0, wsem1, wsem2, wsem3)
    gsems = (gsem0, gsem1, gsem2, gsem3)

    wid = lax.axis_index("s") * 2 + lax.axis_index("c")
    base_s = wid * SPW

    pltpu.sync_copy(x_hbm.at[pl.ds(base_s, SPW)], x_v)
    pltpu.sync_copy(table_hbm, table_v)

    iota = lax.iota(jnp.int32, L)
    ones_v = _vfull(1)
    zeros_v = _vfull(0)
    negones_v = _vfull(-1)

    def build_sample(smp, smp_off):
        acc = zeros_v
        for c in range(N_CHUNKS):
            acc = acc + x_v[smp, pl.ds(c * L, L)]
        tot = _bcast_last(plsc.cumsum(acc + _vfull(7))) - _vfull(7 * L)
        n_up = lax.shift_right_arithmetic(tot + _vfull(N_SITES), ones_v)
        carry = zeros_v
        for c in range(N_CHUNKS):
            v = x_v[smp, pl.ds(c * L, L)]
            up = v == ones_v
            dn = v == negones_v
            upi = jnp.where(up, ones_v, zeros_v)
            cs = plsc.cumsum(upi)
            up_ex = carry + cs - upi
            i_loc = iota + _vfull(c * L)
            p = jnp.where(up, up_ex, n_up + i_loc - up_ex)
            src = jnp.where(dn, i_loc + _vfull(N_SITES), i_loc)
            if c < 6:
                plsc.store_scatter(idx1d, [smp_off + p], src)
            else:
                plsc.store_scatter(idx1d, [smp_off + p], src,
                                   mask=i_loc < _vfull(IDX_STRIDE))
            carry = carry + _bcast_last(cs)

    def assemble_sample(smp, b):
        base = smp * IDX_STRIDE

        def grp(g, _):
            j0 = jnp.minimum(g * L, IDX_STRIDE - L)
            rv = idx1d[pl.ds(base + j0, L)]
            for l in range(L):
                r = rv[l]
                for c in range(DCH):
                    rows[b][j0 + l, pl.ds(c * L, L)] = \
                        table_v[r, pl.ds(c * L, L)]
            return 0

        lax.fori_loop(G_GRP0, N_CHUNKS, grp, 0)

    def gather(k, b):
        pltpu.async_copy(
            table_hbm.at[idx1d.at[pl.ds(k * IDX_STRIDE, G_ROWS)]],
            rows[b].at[pl.ds(0, G_ROWS)], gsems[b])

    def wait_gather(b):
        pltpu.make_async_copy(
            table_hbm.at[idx1d.at[pl.ds(0, G_ROWS)]],
            rows[b].at[pl.ds(0, G_ROWS)], gsems[b]).wait()

    def write(k, b):
        pltpu.async_copy(
            rows[b].at[pl.ds(0, N_SITES)], out_hbm.at[base_s + k], wsems[b])

    def wait_write(b):
        pltpu.make_async_copy(
            rows[b].at[pl.ds(0, N_SITES)], out_hbm.at[base_s], wsems[b]).wait()

    def move(g, off0):
        for b in range(NBUF):
            k = g * NBUF + b
            build_sample(k, off0 + _vfull(b * IDX_STRIDE))

            @pl.when(g >= 1)
            def _():
                wait_write(b)

            gather(k, b)
            assemble_sample(k, b)
            wait_gather(b)
            write(k, b)
        return off0 + _vfull(NBUF * IDX_STRIDE)

    lax.fori_loop(0, SPW // NBUF, move, zeros_v)

    for b in range(NBUF):
        wait_write(b)


_sc_kernel = functools.partial(
    pl.kernel,
    out_type=jax.ShapeDtypeStruct((N_SAMPLES, N_SITES, D), jnp.float32),
    mesh=plsc.VectorSubcoreMesh(core_axis_name="c", subcore_axis_name="s"),
    compiler_params=pltpu.CompilerParams(
        needs_layout_passes=False, use_tc_tiling_on_sc=True),
    scratch_types=[
        pltpu.VMEM((SPW, SITES_PAD), jnp.int32),
        pltpu.VMEM((N_ROWS, D), jnp.float32),
        pltpu.VMEM((SPW * IDX_STRIDE + L,), jnp.int32),
    ]
    + [pltpu.VMEM((ROWS_BUF, D), jnp.float32) for _ in range(NBUF)]
    + [pltpu.SemaphoreType.DMA for _ in range(2 * NBUF)],
)(_sc_body)


def kernel(x, orbitals_mf, orbitals_hf):
    n_samples, n_sites = x.shape
    assert (n_samples, n_sites) == (N_SAMPLES, N_SITES)
    table = jnp.concatenate([orbitals_mf, orbitals_hf], axis=1)
    xp = jnp.pad(x.astype(jnp.int32), ((0, 0), (0, SITES_PAD - n_sites)))
    return _sc_kernel(xp, table)


# speedup vs baseline: 1.5977x; 1.1455x over previous
import functools

import jax
import jax.numpy as jnp
from jax import lax
from jax.experimental import pallas as pl
from jax.experimental.pallas import tpu as pltpu
from jax.experimental.pallas import tpu_sc as plsc

L = 16
NW = 32
N_SAMPLES = 4096
N_SITES = 100
N_ROWS = 2 * N_SITES
D = 128
SITES_PAD = 128
N_CHUNKS = 7
ROWS_BUF = 104
DCH = D // L
SPW = N_SAMPLES // NW
IDX_STRIDE = 104
NBUF = 4
G_ROWS = 48
G_GRP0 = G_ROWS // L


def _vfull(val):
    return jnp.full((L,), val, jnp.int32)


def _bcast_last(cs):
    return plsc.cummax(lax.rev(cs, (0,)))


def _sc_body(x_hbm, table_hbm, out_hbm, x_v, table_v, idx1d,
             rows0, rows1, rows2, rows3,
             wsem0, wsem1, wsem2, wsem3,
             gsem0, gsem1, gsem2, gsem3):
    rows = (rows0, rows1, rows2, rows3)
    wsems = (wsem0, wsem1, wsem2, wsem3)
    gsems = (gsem0, gsem1, gsem2, gsem3)

    wid = lax.axis_index("s") * 2 + lax.axis_index("c")
    base_s = wid * SPW

    pltpu.sync_copy(x_hbm.at[pl.ds(base_s, SPW)], x_v)
    pltpu.sync_copy(table_hbm, table_v)

    iota = lax.iota(jnp.int32, L)
    ones_v = _vfull(1)
    zeros_v = _vfull(0)
    negones_v = _vfull(-1)

    def build_sample(smp, smp_off):
        acc = zeros_v
        for c in range(N_CHUNKS):
            acc = acc + x_v[smp, pl.ds(c * L, L)]
        tot = _bcast_last(plsc.cumsum(acc + _vfull(7))) - _vfull(7 * L)
        n_up = lax.shift_right_arithmetic(tot + _vfull(N_SITES), ones_v)
        carry = zeros_v
        for c in range(N_CHUNKS):
            v = x_v[smp, pl.ds(c * L, L)]
            up = v == ones_v
            dn = v == negones_v
            upi = jnp.where(up, ones_v, zeros_v)
            cs = plsc.cumsum(upi)
            up_ex = carry + cs - upi
            i_loc = iota + _vfull(c * L)
            p = jnp.where(up, up_ex, n_up + i_loc - up_ex)
            src = jnp.where(dn, i_loc + _vfull(N_SITES), i_loc)
            if c < 6:
                plsc.store_scatter(idx1d, [smp_off + p], src)
            else:
                plsc.store_scatter(idx1d, [smp_off + p], src,
                                   mask=i_loc < _vfull(IDX_STRIDE))
            carry = carry + _bcast_last(cs)

    def assemble_sample(smp, b):
        base = smp * IDX_STRIDE

        def grp(g, _):
            j0 = jnp.minimum(g * L, IDX_STRIDE - L)
            rv = idx1d[pl.ds(base + j0, L)]
            for l in range(L):
                r = rv[l]
                for c in range(DCH):
                    rows[b][j0 + l, pl.ds(c * L, L)] = \
                        table_v[r, pl.ds(c * L, L)]
            return 0

        lax.fori_loop(G_GRP0, N_CHUNKS, grp, 0)

    def gather(k, b):
        pltpu.async_copy(
            table_hbm.at[idx1d.at[pl.ds(k * IDX_STRIDE, G_ROWS)]],
            rows[b].at[pl.ds(0, G_ROWS)], gsems[b])

    def wait_gather(b):
        pltpu.make_async_copy(
            table_hbm.at[idx1d.at[pl.ds(0, G_ROWS)]],
            rows[b].at[pl.ds(0, G_ROWS)], gsems[b]).wait()

    def write(k, b):
        pltpu.async_copy(
            rows[b].at[pl.ds(0, N_SITES)], out_hbm.at[base_s + k], wsems[b])

    def wait_write(b):
        pltpu.make_async_copy(
            rows[b].at[pl.ds(0, N_SITES)], out_hbm.at[base_s], wsems[b]).wait()

    def move(g, off0):
        for b in range(NBUF):
            k = g * NBUF + b
            build_sample(k, off0 + _vfull(b * IDX_STRIDE))

            @pl.when(g >= 1)
            def _():
                wait_write(b)

            gather(k, b)
            assemble_sample(k, b)
            wait_gather(b)
            write(k, b)
        return off0 + _vfull(NBUF * IDX_STRIDE)

    lax.fori_loop(0, SPW // NBUF, move, zeros_v)

    for b in range(NBUF):
        wait_write(b)


_sc_kernel = functools.partial(
    pl.kernel,
    out_type=jax.ShapeDtypeStruct((N_SAMPLES, N_SITES, D), jnp.float32),
    mesh=plsc.VectorSubcoreMesh(core_axis_name="c", subcore_axis_name="s"),
    compiler_params=pltpu.CompilerParams(
        needs_layout_passes=False, use_tc_tiling_on_sc=True),
    scratch_types=[
        pltpu.VMEM((SPW, SITES_PAD), jnp.int32),
        pltpu.VMEM((N_ROWS, D), jnp.float32),
        pltpu.VMEM((SPW * IDX_STRIDE + L,), jnp.int32),
    ]
    + [pltpu.VMEM((ROWS_BUF, D), jnp.float32) for _ in range(NBUF)]
    + [pltpu.SemaphoreType.DMA for _ in range(2 * NBUF)],
)(_sc_body)


def kernel(x, orbitals_mf, orbitals_hf):
    n_samples, n_sites = x.shape
    assert (n_samples, n_sites) == (N_SAMPLES, N_SITES)
    table = jnp.concatenate([orbitals_mf, orbitals_hf], axis=1)
    xp = jnp.pad(x.astype(jnp.int32), ((0, 0), (0, SITES_PAD - n_sites)))
    return _sc_kernel(xp, table)
